# Initial kernel scaffold; baseline (speedup 1.0000x reference)
import jax
import jax.numpy as jnp
from jax.experimental import pallas as pl


def kernel(x, adj, W, b):
    raise NotImplementedError("write your pallas kernel here")


# speedup vs baseline: 5630.4150x; 5630.4150x over previous
import jax
import jax.numpy as jnp
from jax.experimental import pallas as pl

_N = 2048
_F = 16
_CHUNK = 256
_NCHUNK = _N // _CHUNK


def _gcn_kernel(x_ref, adj_ref, w_ref, b_ref, out_ref):
    n = adj_ref.shape[0]
    ones_col = jnp.ones((_CHUNK, 1), jnp.float32)

    def mask_chunk(i):
        a = adj_ref[pl.ds(i * _CHUNK, _CHUNK), :]
        rowid = jax.lax.broadcasted_iota(jnp.int32, (_CHUNK, n), 0) + i * _CHUNK
        colid = jax.lax.broadcasted_iota(jnp.int32, (_CHUNK, n), 1)
        return jnp.where((a != 0.0) | (rowid == colid), 1.0, 0.0)

    deg = jnp.zeros((n, 1), jnp.float32)
    for i in range(_NCHUNK):
        deg = deg + jax.lax.dot_general(
            mask_chunk(i), ones_col, (((0,), (0,)), ((), ())),
            preferred_element_type=jnp.float32)
    dinv = jnp.where(deg > 0.0, jax.lax.rsqrt(deg), 0.0)

    h = jax.lax.dot_general(x_ref[...], w_ref[...],
                            (((1,), (1,)), ((), ())),
                            preferred_element_type=jnp.float32)
    g = dinv * h

    s = jnp.zeros((n, _F), jnp.float32)
    for i in range(_NCHUNK):
        gi = jax.lax.slice(g, (i * _CHUNK, 0), ((i + 1) * _CHUNK, _F))
        s = s + jax.lax.dot_general(
            mask_chunk(i), gi, (((0,), (0,)), ((), ())),
            preferred_element_type=jnp.float32)
    out_ref[...] = dinv * s + b_ref[...]


def kernel(x, adj, W, b):
    return pl.pallas_call(
        _gcn_kernel,
        out_shape=jax.ShapeDtypeStruct((_N, _F), jnp.float32),
    )(x, adj, W, b.reshape(1, _F))


# speedup vs baseline: 6204.2186x; 1.1019x over previous
import jax
import jax.numpy as jnp
from jax.experimental import pallas as pl
from jax.experimental.pallas import tpu as pltpu

_N = 2048
_F = 16
_CHUNK = 256
_NBLK = _N // _CHUNK


def _gcn_kernel(x_ref, adj_ref, w_ref, b_ref, out_ref, mask_s, deg_s):
    i = pl.program_id(0)
    a = adj_ref[...]
    rowid = jax.lax.broadcasted_iota(jnp.int32, (_CHUNK, _N), 0) + i * _CHUNK
    colid = jax.lax.broadcasted_iota(jnp.int32, (_CHUNK, _N), 1)
    m = jnp.where((a != 0.0) | (rowid == colid),
                  1.0, 0.0).astype(jnp.bfloat16)
    mask_s[i] = m

    ones_col = jnp.ones((_CHUNK, 1), jnp.bfloat16)
    dcontrib = jax.lax.dot_general(m, ones_col, (((0,), (0,)), ((), ())),
                                   preferred_element_type=jnp.float32)

    @pl.when(i == 0)
    def _init():
        deg_s[...] = dcontrib

    @pl.when(i > 0)
    def _acc():
        deg_s[...] = deg_s[...] + dcontrib

    @pl.when(i == _NBLK - 1)
    def _finish():
        deg = deg_s[...]
        dinv = jnp.where(deg > 0.0, jax.lax.rsqrt(deg), 0.0)
        h = jax.lax.dot_general(x_ref[...], w_ref[...],
                                (((1,), (1,)), ((), ())),
                                preferred_element_type=jnp.float32)
        g = (dinv * h).astype(jnp.bfloat16)
        s = jnp.zeros((_N, _F), jnp.float32)
        for k in range(_NBLK):
            gk = jax.lax.slice(g, (k * _CHUNK, 0), ((k + 1) * _CHUNK, _F))
            s = s + jax.lax.dot_general(
                mask_s[k], gk, (((0,), (0,)), ((), ())),
                preferred_element_type=jnp.float32)
        out_ref[...] = dinv * s + b_ref[...]


def kernel(x, adj, W, b):
    return pl.pallas_call(
        _gcn_kernel,
        grid=(_NBLK,),
        in_specs=[
            pl.BlockSpec((_N, _F), lambda i: (0, 0)),
            pl.BlockSpec((_CHUNK, _N), lambda i: (i, 0)),
            pl.BlockSpec((_F, _F), lambda i: (0, 0)),
            pl.BlockSpec((1, _F), lambda i: (0, 0)),
        ],
        out_specs=pl.BlockSpec((_N, _F), lambda i: (0, 0)),
        scratch_shapes=[
            pltpu.VMEM((_NBLK, _CHUNK, _N), jnp.bfloat16),
            pltpu.VMEM((_N, 1), jnp.float32),
        ],
        out_shape=jax.ShapeDtypeStruct((_N, _F), jnp.float32),
    )(x, adj, W, b.reshape(1, _F))


# speedup vs baseline: 6923.8896x; 1.1160x over previous
import jax
import jax.numpy as jnp
from jax.experimental import pallas as pl
from jax.experimental.pallas import tpu as pltpu

_N = 2048
_F = 16
_CHUNK = 512
_NBLK = _N // _CHUNK


def _gcn_kernel(x_ref, adj_ref, w_ref, b_ref, out_ref, mask_s, deg_s):
    i = pl.program_id(0)
    a = adj_ref[...]
    rowid = jax.lax.broadcasted_iota(jnp.int32, (_CHUNK, _N), 0) + i * _CHUNK
    colid = jax.lax.broadcasted_iota(jnp.int32, (_CHUNK, _N), 1)
    m = jnp.where((a != 0.0) | (rowid == colid),
                  1.0, 0.0).astype(jnp.bfloat16)
    mask_s[i] = m

    ones_col = jnp.ones((_CHUNK, 1), jnp.bfloat16)
    dcontrib = jax.lax.dot_general(m, ones_col, (((0,), (0,)), ((), ())),
                                   preferred_element_type=jnp.float32)

    @pl.when(i == 0)
    def _init():
        deg_s[...] = dcontrib

    @pl.when(i > 0)
    def _acc():
        deg_s[...] = deg_s[...] + dcontrib

    @pl.when(i == _NBLK - 1)
    def _finish():
        deg = deg_s[...]
        dinv = jnp.where(deg > 0.0, jax.lax.rsqrt(deg), 0.0)
        h = jax.lax.dot_general(x_ref[...], w_ref[...],
                                (((1,), (1,)), ((), ())),
                                preferred_element_type=jnp.float32)
        g = (dinv * h).astype(jnp.bfloat16)
        s = jnp.zeros((_N, _F), jnp.float32)
        for k in range(_NBLK):
            gk = jax.lax.slice(g, (k * _CHUNK, 0), ((k + 1) * _CHUNK, _F))
            s = s + jax.lax.dot_general(
                mask_s[k], gk, (((0,), (0,)), ((), ())),
                preferred_element_type=jnp.float32)
        out_ref[...] = dinv * s + b_ref[...]


def kernel(x, adj, W, b):
    return pl.pallas_call(
        _gcn_kernel,
        grid=(_NBLK,),
        in_specs=[
            pl.BlockSpec((_N, _F), lambda i: (0, 0)),
            pl.BlockSpec((_CHUNK, _N), lambda i: (i, 0)),
            pl.BlockSpec((_F, _F), lambda i: (0, 0)),
            pl.BlockSpec((1, _F), lambda i: (0, 0)),
        ],
        out_specs=pl.BlockSpec((_N, _F), lambda i: (0, 0)),
        scratch_shapes=[
            pltpu.VMEM((_NBLK, _CHUNK, _N), jnp.bfloat16),
            pltpu.VMEM((_N, 1), jnp.float32),
        ],
        out_shape=jax.ShapeDtypeStruct((_N, _F), jnp.float32),
    )(x, adj, W, b.reshape(1, _F))


# speedup vs baseline: 7051.3870x; 1.0184x over previous
import jax
import jax.numpy as jnp
from jax.experimental import pallas as pl
from jax.experimental.pallas import tpu as pltpu

_N = 2048
_F = 16
_CHUNK = 1024
_NBLK = _N // _CHUNK


def _gcn_kernel(x_ref, adj_ref, w_ref, b_ref, out_ref, mask_s, deg_s):
    i = pl.program_id(0)
    a = adj_ref[...]
    rowid = jax.lax.broadcasted_iota(jnp.int32, (_CHUNK, _N), 0) + i * _CHUNK
    colid = jax.lax.broadcasted_iota(jnp.int32, (_CHUNK, _N), 1)
    m = jnp.where((a != 0.0) | (rowid == colid),
                  1.0, 0.0).astype(jnp.bfloat16)
    mask_s[i] = m

    ones_col = jnp.ones((_CHUNK, 1), jnp.bfloat16)
    dcontrib = jax.lax.dot_general(m, ones_col, (((0,), (0,)), ((), ())),
                                   preferred_element_type=jnp.float32)

    @pl.when(i == 0)
    def _init():
        deg_s[...] = dcontrib

    @pl.when(i > 0)
    def _acc():
        deg_s[...] = deg_s[...] + dcontrib

    @pl.when(i == _NBLK - 1)
    def _finish():
        deg = deg_s[...]
        dinv = jnp.where(deg > 0.0, jax.lax.rsqrt(deg), 0.0)
        h = jax.lax.dot_general(x_ref[...], w_ref[...],
                                (((1,), (1,)), ((), ())),
                                preferred_element_type=jnp.float32)
        g = (dinv * h).astype(jnp.bfloat16)
        s = jnp.zeros((_N, _F), jnp.float32)
        for k in range(_NBLK):
            gk = jax.lax.slice(g, (k * _CHUNK, 0), ((k + 1) * _CHUNK, _F))
            s = s + jax.lax.dot_general(
                mask_s[k], gk, (((0,), (0,)), ((), ())),
                preferred_element_type=jnp.float32)
        out_ref[...] = dinv * s + b_ref[...]


def kernel(x, adj, W, b):
    return pl.pallas_call(
        _gcn_kernel,
        grid=(_NBLK,),
        in_specs=[
            pl.BlockSpec((_N, _F), lambda i: (0, 0)),
            pl.BlockSpec((_CHUNK, _N), lambda i: (i, 0)),
            pl.BlockSpec((_F, _F), lambda i: (0, 0)),
            pl.BlockSpec((1, _F), lambda i: (0, 0)),
        ],
        out_specs=pl.BlockSpec((_N, _F), lambda i: (0, 0)),
        scratch_shapes=[
            pltpu.VMEM((_NBLK, _CHUNK, _N), jnp.bfloat16),
            pltpu.VMEM((_N, 1), jnp.float32),
        ],
        out_shape=jax.ShapeDtypeStruct((_N, _F), jnp.float32),
    )(x, adj, W, b.reshape(1, _F))
